# Initial kernel scaffold; baseline (speedup 1.0000x reference)
import jax
import jax.numpy as jnp
from jax.experimental import pallas as pl


def kernel(b):
    raise NotImplementedError("write your pallas kernel here")


# speedup vs baseline: 1119.3713x; 1119.3713x over previous
import functools

import jax
import jax.numpy as jnp
import numpy as np
from jax import lax
from jax.experimental import pallas as pl
from jax.experimental.pallas import tpu as pltpu
from jax.experimental.pallas import tpu_sc as plsc

_KNOTS = np.array(
    [[0.0, 0.0], [0.0625, 0.382683], [0.125, 0.707107], [0.1875, 0.92388],
     [0.25, 1.0], [0.3125, 0.92388], [0.375, 0.707107], [0.4375, 0.382683],
     [0.5, 0.0], [0.5625, -0.382683], [0.625, -0.707107], [0.6875, -0.92388],
     [0.75, -1.0], [0.8125, -0.92388], [0.875, -0.707107],
     [0.9375, -0.382683], [1.0, 0.0]], dtype=np.float64)


def _spline_coeffs(x, y):
    x = np.asarray(x, dtype=np.float64)
    y = np.asarray(y, dtype=np.float64)
    n = len(x)
    dx = np.diff(x)
    slope = np.diff(y) / dx
    A = np.zeros((n, n))
    rhs = np.zeros(n)
    d0 = x[2] - x[0]
    A[0, 0] = dx[1]
    A[0, 1] = d0
    rhs[0] = ((dx[0] + 2.0 * d0) * dx[1] * slope[0] + dx[0] ** 2 * slope[1]) / d0
    for i in range(1, n - 1):
        A[i, i - 1] = dx[i]
        A[i, i] = 2.0 * (dx[i - 1] + dx[i])
        A[i, i + 1] = dx[i - 1]
        rhs[i] = 3.0 * (dx[i] * slope[i - 1] + dx[i - 1] * slope[i])
    dn = x[-1] - x[-3]
    A[-1, -1] = dx[-2]
    A[-1, -2] = dn
    rhs[-1] = ((dx[-1] + 2.0 * dn) * dx[-2] * slope[-1]
               + dx[-1] ** 2 * slope[-2]) / dn
    s = np.linalg.solve(A, rhs)
    t = (s[:-1] + s[1:] - 2.0 * slope) / dx
    c = np.zeros((4, n - 1))
    c[0] = t / dx
    c[1] = (slope - s[:-1]) / dx - t
    c[2] = s[:-1]
    c[3] = y[:-1]
    return c

_COEF = np.asarray(_spline_coeffs(_KNOTS[:, 0], _KNOTS[:, 1]),
                   dtype=np.float32)

_N = 8388608
_NW = 32
_PER_W = _N // _NW
_CHUNK = 32768
_NCHUNK = _PER_W // _CHUNK
_L = 16
_NSEG = 15
_INV_DIS = 16.0
_DIS = 0.0625


def _spline_body(x_hbm, tab_hbm, out_hbm, tab_v, xbuf, obuf):
    cid = lax.axis_index("c")
    sid = lax.axis_index("s")
    wid = sid * 2 + cid
    base = wid * _PER_W
    pltpu.sync_copy(tab_hbm, tab_v)
    c0r = tab_v[pl.ds(0, _L)]
    c1r = tab_v[pl.ds(_L, _L)]
    c2r = tab_v[pl.ds(2 * _L, _L)]
    c3r = tab_v[pl.ds(3 * _L, _L)]

    def chunk_body(ci, carry):
        off = base + ci * _CHUNK
        pltpu.sync_copy(x_hbm.at[pl.ds(off, _CHUNK)], xbuf)

        def vec_body(vi, carry2):
            x = xbuf[pl.ds(vi * _L, _L)]
            l = jnp.minimum((x * _INV_DIS + 1e-05).astype(jnp.int32), _NSEG)
            bx = x - l.astype(jnp.float32) * _DIS
            c0 = _vgather(c0r, l)
            c1 = _vgather(c1r, l)
            c2 = _vgather(c2r, l)
            c3 = _vgather(c3r, l)
            v = c3 + bx * (c2 + bx * (c1 + bx * c0))
            obuf[pl.ds(vi * _L, _L)] = v
            return carry2

        lax.fori_loop(0, _CHUNK // _L, vec_body, 0, unroll=4)
        pltpu.sync_copy(obuf, out_hbm.at[pl.ds(off, _CHUNK)])
        return carry

    lax.fori_loop(0, _NCHUNK, chunk_body, 0)


_GATHER_DNUMS = lax.GatherDimensionNumbers(
    offset_dims=(), collapsed_slice_dims=(0,), start_index_map=(0,))


def _vgather(row, l):
    return lax.gather(row, l[:, None], _GATHER_DNUMS, (1,),
                      mode=lax.GatherScatterMode.PROMISE_IN_BOUNDS)


_mesh = plsc.VectorSubcoreMesh(core_axis_name="c", subcore_axis_name="s")

_spline_call = functools.partial(
    pl.kernel,
    mesh=_mesh,
    out_type=jax.ShapeDtypeStruct((_N,), jnp.float32),
    scratch_types=[
        pltpu.VMEM((4 * _L,), jnp.float32),
        pltpu.VMEM((_CHUNK,), jnp.float32),
        pltpu.VMEM((_CHUNK,), jnp.float32),
    ],
)(_spline_body)


@jax.jit
def kernel(b):
    x = b.reshape(_N)
    tab = jnp.asarray(_COEF.reshape(-1))
    return _spline_call(x, tab)


# speedup vs baseline: 1172.3052x; 1.0473x over previous
import functools

import jax
import jax.numpy as jnp
import numpy as np
from jax import lax
from jax.experimental import pallas as pl
from jax.experimental.pallas import tpu as pltpu
from jax.experimental.pallas import tpu_sc as plsc

_KNOTS = np.array(
    [[0.0, 0.0], [0.0625, 0.382683], [0.125, 0.707107], [0.1875, 0.92388],
     [0.25, 1.0], [0.3125, 0.92388], [0.375, 0.707107], [0.4375, 0.382683],
     [0.5, 0.0], [0.5625, -0.382683], [0.625, -0.707107], [0.6875, -0.92388],
     [0.75, -1.0], [0.8125, -0.92388], [0.875, -0.707107],
     [0.9375, -0.382683], [1.0, 0.0]], dtype=np.float64)


def _spline_coeffs(x, y):
    x = np.asarray(x, dtype=np.float64)
    y = np.asarray(y, dtype=np.float64)
    n = len(x)
    dx = np.diff(x)
    slope = np.diff(y) / dx
    A = np.zeros((n, n))
    rhs = np.zeros(n)
    d0 = x[2] - x[0]
    A[0, 0] = dx[1]
    A[0, 1] = d0
    rhs[0] = ((dx[0] + 2.0 * d0) * dx[1] * slope[0] + dx[0] ** 2 * slope[1]) / d0
    for i in range(1, n - 1):
        A[i, i - 1] = dx[i]
        A[i, i] = 2.0 * (dx[i - 1] + dx[i])
        A[i, i + 1] = dx[i - 1]
        rhs[i] = 3.0 * (dx[i] * slope[i - 1] + dx[i - 1] * slope[i])
    dn = x[-1] - x[-3]
    A[-1, -1] = dx[-2]
    A[-1, -2] = dn
    rhs[-1] = ((dx[-1] + 2.0 * dn) * dx[-2] * slope[-1]
               + dx[-1] ** 2 * slope[-2]) / dn
    s = np.linalg.solve(A, rhs)
    t = (s[:-1] + s[1:] - 2.0 * slope) / dx
    c = np.zeros((4, n - 1))
    c[0] = t / dx
    c[1] = (slope - s[:-1]) / dx - t
    c[2] = s[:-1]
    c[3] = y[:-1]
    return c

_COEF = np.asarray(_spline_coeffs(_KNOTS[:, 0], _KNOTS[:, 1]),
                   dtype=np.float32)

_N = 8388608
_NW = 32
_PER_W = _N // _NW
_CHUNK = 16384
_NCHUNK = _PER_W // _CHUNK
_L = 16
_NSEG = 15
_INV_DIS = 16.0
_DIS = 0.0625

_GATHER_DNUMS = lax.GatherDimensionNumbers(
    offset_dims=(), collapsed_slice_dims=(0,), start_index_map=(0,))


def _vgather(row, l):
    return lax.gather(row, l[:, None], _GATHER_DNUMS, (1,),
                      mode=lax.GatherScatterMode.PROMISE_IN_BOUNDS)


def _spline_body(x_hbm, tab_hbm, out_hbm, tab_v,
                 xb0, xb1, ob0, ob1, ls0, ls1, ss0, ss1):
    cid = lax.axis_index("c")
    sid = lax.axis_index("s")
    wid = sid * 2 + cid
    base = wid * _PER_W
    pltpu.sync_copy(tab_hbm, tab_v)
    c0r = tab_v[pl.ds(0, _L)]
    c1r = tab_v[pl.ds(_L, _L)]
    c2r = tab_v[pl.ds(2 * _L, _L)]
    c3r = tab_v[pl.ds(3 * _L, _L)]

    def compute(xbuf, obuf):
        def vec_body(vi, carry):
            x = xbuf[pl.ds(vi * _L, _L)]
            l = jnp.minimum((x * _INV_DIS + 1e-05).astype(jnp.int32), _NSEG)
            bx = x - l.astype(jnp.float32) * _DIS
            c0 = _vgather(c0r, l)
            c1 = _vgather(c1r, l)
            c2 = _vgather(c2r, l)
            c3 = _vgather(c3r, l)
            v = c3 + bx * (c2 + bx * (c1 + bx * c0))
            obuf[pl.ds(vi * _L, _L)] = v
            return carry

        lax.fori_loop(0, _CHUNK // _L, vec_body, 0, unroll=8)

    xbufs = [xb0, xb1]
    obufs = [ob0, ob1]
    lsems = [ls0, ls1]
    ssems = [ss0, ss1]
    loads = [None, None]
    stores = [None, None]
    loads[0] = pltpu.async_copy(x_hbm.at[pl.ds(base, _CHUNK)], xb0, ls0)
    for ci in range(_NCHUNK):
        cur = ci & 1
        nxt = 1 - cur
        if ci + 1 < _NCHUNK:
            off_n = base + (ci + 1) * _CHUNK
            loads[nxt] = pltpu.async_copy(
                x_hbm.at[pl.ds(off_n, _CHUNK)], xbufs[nxt], lsems[nxt])
        loads[cur].wait()
        if stores[cur] is not None:
            stores[cur].wait()
        compute(xbufs[cur], obufs[cur])
        stores[cur] = pltpu.async_copy(
            obufs[cur], out_hbm.at[pl.ds(base + ci * _CHUNK, _CHUNK)],
            ssems[cur])
    stores[0].wait()
    stores[1].wait()


_mesh = plsc.VectorSubcoreMesh(core_axis_name="c", subcore_axis_name="s")

_spline_call = functools.partial(
    pl.kernel,
    mesh=_mesh,
    out_type=jax.ShapeDtypeStruct((_N,), jnp.float32),
    scratch_types=[
        pltpu.VMEM((4 * _L,), jnp.float32),
        pltpu.VMEM((_CHUNK,), jnp.float32),
        pltpu.VMEM((_CHUNK,), jnp.float32),
        pltpu.VMEM((_CHUNK,), jnp.float32),
        pltpu.VMEM((_CHUNK,), jnp.float32),
        pltpu.SemaphoreType.DMA,
        pltpu.SemaphoreType.DMA,
        pltpu.SemaphoreType.DMA,
        pltpu.SemaphoreType.DMA,
    ],
)(_spline_body)


@jax.jit
def kernel(b):
    x = b.reshape(_N)
    tab = jnp.asarray(_COEF.reshape(-1))
    return _spline_call(x, tab)


# speedup vs baseline: 4591.0172x; 3.9162x over previous
import functools

import jax
import jax.numpy as jnp
import numpy as np
from jax import lax
from jax.experimental import pallas as pl
from jax.experimental.pallas import tpu as pltpu
from jax.experimental.pallas import tpu_sc as plsc

_KNOTS = np.array(
    [[0.0, 0.0], [0.0625, 0.382683], [0.125, 0.707107], [0.1875, 0.92388],
     [0.25, 1.0], [0.3125, 0.92388], [0.375, 0.707107], [0.4375, 0.382683],
     [0.5, 0.0], [0.5625, -0.382683], [0.625, -0.707107], [0.6875, -0.92388],
     [0.75, -1.0], [0.8125, -0.92388], [0.875, -0.707107],
     [0.9375, -0.382683], [1.0, 0.0]], dtype=np.float64)


def _spline_coeffs(x, y):
    x = np.asarray(x, dtype=np.float64)
    y = np.asarray(y, dtype=np.float64)
    n = len(x)
    dx = np.diff(x)
    slope = np.diff(y) / dx
    A = np.zeros((n, n))
    rhs = np.zeros(n)
    d0 = x[2] - x[0]
    A[0, 0] = dx[1]
    A[0, 1] = d0
    rhs[0] = ((dx[0] + 2.0 * d0) * dx[1] * slope[0] + dx[0] ** 2 * slope[1]) / d0
    for i in range(1, n - 1):
        A[i, i - 1] = dx[i]
        A[i, i] = 2.0 * (dx[i - 1] + dx[i])
        A[i, i + 1] = dx[i - 1]
        rhs[i] = 3.0 * (dx[i] * slope[i - 1] + dx[i - 1] * slope[i])
    dn = x[-1] - x[-3]
    A[-1, -1] = dx[-2]
    A[-1, -2] = dn
    rhs[-1] = ((dx[-1] + 2.0 * dn) * dx[-2] * slope[-1]
               + dx[-1] ** 2 * slope[-2]) / dn
    s = np.linalg.solve(A, rhs)
    t = (s[:-1] + s[1:] - 2.0 * slope) / dx
    c = np.zeros((4, n - 1))
    c[0] = t / dx
    c[1] = (slope - s[:-1]) / dx - t
    c[2] = s[:-1]
    c[3] = y[:-1]
    return c

_COEF = np.asarray(_spline_coeffs(_KNOTS[:, 0], _KNOTS[:, 1]),
                   dtype=np.float32)

_N = 8388608
_NW = 32
_PER_W = _N // _NW
_CHUNK = 16384
_NCHUNK = _PER_W // _CHUNK
_L = 16
_NSEG = 15
_INV_DIS = 16.0
_DIS = 0.0625

_GATHER_DNUMS = lax.GatherDimensionNumbers(
    offset_dims=(), collapsed_slice_dims=(0,), start_index_map=(0,))


def _vgather(row, l):
    return lax.gather(row, l[:, None], _GATHER_DNUMS, (1,),
                      mode=lax.GatherScatterMode.PROMISE_IN_BOUNDS)


def _spline_body(x_hbm, tab_hbm, out_hbm, tab_v,
                 xb0, xb1, ob0, ob1, ls0, ls1, ss0, ss1):
    cid = lax.axis_index("c")
    sid = lax.axis_index("s")
    wid = sid * 2 + cid
    base = wid * _PER_W
    pltpu.sync_copy(tab_hbm, tab_v)
    c0r = tab_v[pl.ds(0, _L)]
    c1r = tab_v[pl.ds(_L, _L)]
    c2r = tab_v[pl.ds(2 * _L, _L)]
    c3r = tab_v[pl.ds(3 * _L, _L)]

    def compute(xbuf, obuf):
        @plsc.parallel_loop(0, _CHUNK, step=_L, unroll=8)
        def vec_body(i):
            x = xbuf[pl.ds(i, _L)]
            l = jnp.minimum((x * _INV_DIS + 1e-05).astype(jnp.int32), _NSEG)
            bx = x - l.astype(jnp.float32) * _DIS
            c0 = _vgather(c0r, l)
            c1 = _vgather(c1r, l)
            c2 = _vgather(c2r, l)
            c3 = _vgather(c3r, l)
            v = c3 + bx * (c2 + bx * (c1 + bx * c0))
            obuf[pl.ds(i, _L)] = v

    xbufs = [xb0, xb1]
    obufs = [ob0, ob1]
    lsems = [ls0, ls1]
    ssems = [ss0, ss1]
    loads = [None, None]
    stores = [None, None]
    loads[0] = pltpu.async_copy(x_hbm.at[pl.ds(base, _CHUNK)], xb0, ls0)
    for ci in range(_NCHUNK):
        cur = ci & 1
        nxt = 1 - cur
        if ci + 1 < _NCHUNK:
            off_n = base + (ci + 1) * _CHUNK
            loads[nxt] = pltpu.async_copy(
                x_hbm.at[pl.ds(off_n, _CHUNK)], xbufs[nxt], lsems[nxt])
        loads[cur].wait()
        if stores[cur] is not None:
            stores[cur].wait()
        compute(xbufs[cur], obufs[cur])
        stores[cur] = pltpu.async_copy(
            obufs[cur], out_hbm.at[pl.ds(base + ci * _CHUNK, _CHUNK)],
            ssems[cur])
    stores[0].wait()
    stores[1].wait()


_mesh = plsc.VectorSubcoreMesh(core_axis_name="c", subcore_axis_name="s")

_spline_call = functools.partial(
    pl.kernel,
    mesh=_mesh,
    out_type=jax.ShapeDtypeStruct((_N,), jnp.float32),
    scratch_types=[
        pltpu.VMEM((4 * _L,), jnp.float32),
        pltpu.VMEM((_CHUNK,), jnp.float32),
        pltpu.VMEM((_CHUNK,), jnp.float32),
        pltpu.VMEM((_CHUNK,), jnp.float32),
        pltpu.VMEM((_CHUNK,), jnp.float32),
        pltpu.SemaphoreType.DMA,
        pltpu.SemaphoreType.DMA,
        pltpu.SemaphoreType.DMA,
        pltpu.SemaphoreType.DMA,
    ],
)(_spline_body)


@jax.jit
def kernel(b):
    x = b.reshape(_N)
    tab = jnp.asarray(_COEF.reshape(-1))
    return _spline_call(x, tab)
